# initial kernel scaffold (unmeasured)
import jax
import jax.numpy as jnp
from jax import lax
from jax.experimental import pallas as pl
from jax.experimental.pallas import tpu as pltpu


def kernel(
    x,
):
    def body(*refs):
        pass

    out_shape = jax.ShapeDtypeStruct(..., jnp.float32)
    return pl.pallas_call(body, out_shape=out_shape)(...)


# baseline (device time: 29504 ns/iter reference)
import jax
import jax.numpy as jnp
from jax import lax
from jax.experimental import pallas as pl
from jax.experimental.pallas import tpu as pltpu

N_DEV = 16
N_ROUNDS = 4
BLK = 256


def kernel(x):
    m, n = x.shape
    nblk = m // BLK

    def body(x_ref, out_ref, acc_ref, tot_ref, comm_ref, send_sems, recv_sems):
        my = lax.axis_index("i")

        row = lax.broadcasted_iota(jnp.int32, (BLK, BLK), 0)
        col = lax.broadcasted_iota(jnp.int32, (BLK, BLK), 1)
        tri = (row >= col).astype(jnp.bfloat16)

        carry = jnp.zeros((1, n), jnp.float32)
        for b in range(nblk):
            xb = x_ref[b * BLK:(b + 1) * BLK, :]
            lb = jnp.log(xb)
            cs = lax.dot_general(
                tri, lb.astype(jnp.bfloat16),
                (((1,), (0,)), ((), ())),
                preferred_element_type=jnp.float32,
            )
            out_ref[b * BLK:(b + 1) * BLK, :] = jnp.exp(cs + carry)
            carry = carry + jnp.sum(lb, axis=0, keepdims=True)

        tot_ref[0:1, :] = carry
        acc_ref[0:1, :] = carry

        for r in range(N_ROUNDS):
            d = 1 << r

            @pl.when(my + d < N_DEV)
            def _():
                snd = pltpu.make_async_remote_copy(
                    src_ref=acc_ref,
                    dst_ref=comm_ref.at[r],
                    send_sem=send_sems.at[r],
                    recv_sem=recv_sems.at[r],
                    device_id=(my + d,),
                    device_id_type=pl.DeviceIdType.MESH,
                )
                snd.start()
                snd.wait_send()

            @pl.when(my >= d)
            def _():
                rcv = pltpu.make_async_remote_copy(
                    src_ref=acc_ref,
                    dst_ref=comm_ref.at[r],
                    send_sem=send_sems.at[r],
                    recv_sem=recv_sems.at[r],
                    device_id=(my - d,),
                    device_id_type=pl.DeviceIdType.MESH,
                )
                rcv.wait_recv()
                acc_ref[0:1, :] = acc_ref[0:1, :] + comm_ref[r]

        p = jnp.exp(acc_ref[0:1, :] - tot_ref[0:1, :])
        for b in range(nblk):
            out_ref[b * BLK:(b + 1) * BLK, :] = (
                out_ref[b * BLK:(b + 1) * BLK, :] * p
            )

    return pl.pallas_call(
        body,
        out_shape=jax.ShapeDtypeStruct((m, n), jnp.float32),
        in_specs=[pl.BlockSpec(memory_space=pltpu.VMEM)],
        out_specs=pl.BlockSpec(memory_space=pltpu.VMEM),
        scratch_shapes=[
            pltpu.VMEM((1, n), jnp.float32),
            pltpu.VMEM((1, n), jnp.float32),
            pltpu.VMEM((N_ROUNDS, 1, n), jnp.float32),
            pltpu.SemaphoreType.DMA((N_ROUNDS,)),
            pltpu.SemaphoreType.DMA((N_ROUNDS,)),
        ],
    )(x)


# device time: 29334 ns/iter; 1.0058x vs baseline; 1.0058x over previous
import jax
import jax.numpy as jnp
from jax import lax
from jax.experimental import pallas as pl
from jax.experimental.pallas import tpu as pltpu

N_DEV = 16
N_ROUNDS = 4
BLK = 256


def kernel(x):
    m, n = x.shape
    nblk = m // BLK
    chunk = nblk // N_ROUNDS

    def body(x_ref, out_ref, acc_ref, comm_ref, send_sems, recv_sems):
        my = lax.axis_index("i")

        row = lax.broadcasted_iota(jnp.int32, (BLK, BLK), 0)
        col = lax.broadcasted_iota(jnp.int32, (BLK, BLK), 1)
        tri = (row >= col).astype(jnp.bfloat16)

        def rowprod(v):
            r = v.shape[0]
            while r > 1:
                half = r // 2
                v = v[:half, :] * v[half:2 * half, :]
                r = half
            return v

        t = rowprod(x_ref[0:BLK, :])
        for b in range(1, nblk):
            t = t * rowprod(x_ref[b * BLK:(b + 1) * BLK, :])
        acc_ref[0:1, :] = t

        def compute_block(b, carry):
            xb = x_ref[b * BLK:(b + 1) * BLK, :]
            lb = jnp.log(xb)
            cs = lax.dot_general(
                tri, lb.astype(jnp.bfloat16),
                (((1,), (0,)), ((), ())),
                preferred_element_type=jnp.float32,
            )
            out_ref[b * BLK:(b + 1) * BLK, :] = jnp.exp(cs + carry)
            return carry + jnp.sum(lb, axis=0, keepdims=True)

        carry = jnp.zeros((1, n), jnp.float32)
        for r in range(N_ROUNDS):
            d = 1 << r

            @pl.when(my + d < N_DEV)
            def _():
                snd = pltpu.make_async_remote_copy(
                    src_ref=acc_ref,
                    dst_ref=comm_ref.at[r],
                    send_sem=send_sems.at[r],
                    recv_sem=recv_sems.at[r],
                    device_id=(my + d,),
                    device_id_type=pl.DeviceIdType.MESH,
                )
                snd.start()
                snd.wait_send()

            for b in range(r * chunk, (r + 1) * chunk):
                carry = compute_block(b, carry)

            @pl.when(my >= d)
            def _():
                rcv = pltpu.make_async_remote_copy(
                    src_ref=acc_ref,
                    dst_ref=comm_ref.at[r],
                    send_sem=send_sems.at[r],
                    recv_sem=recv_sems.at[r],
                    device_id=(my - d,),
                    device_id_type=pl.DeviceIdType.MESH,
                )
                rcv.wait_recv()
                acc_ref[0:1, :] = acc_ref[0:1, :] * comm_ref[r]

        for b in range(N_ROUNDS * chunk, nblk):
            carry = compute_block(b, carry)

        p = acc_ref[0:1, :] / t
        for b in range(nblk):
            out_ref[b * BLK:(b + 1) * BLK, :] = (
                out_ref[b * BLK:(b + 1) * BLK, :] * p
            )

    return pl.pallas_call(
        body,
        out_shape=jax.ShapeDtypeStruct((m, n), jnp.float32),
        in_specs=[pl.BlockSpec(memory_space=pltpu.VMEM)],
        out_specs=pl.BlockSpec(memory_space=pltpu.VMEM),
        scratch_shapes=[
            pltpu.VMEM((1, n), jnp.float32),
            pltpu.VMEM((N_ROUNDS, 1, n), jnp.float32),
            pltpu.SemaphoreType.DMA((N_ROUNDS,)),
            pltpu.SemaphoreType.DMA((N_ROUNDS,)),
        ],
    )(x)
